# baseline (speedup 1.0000x reference)
import functools

import jax
import jax.numpy as jnp
from jax.experimental import pallas as pl
from jax.experimental.pallas import tpu as pltpu


def _chamfer_kernel(xa_ref, ya_ref, out_ref, *, nbatch, inv_bn, inv_bm):
    acc = jnp.float32(0.0)
    for b in range(nbatch):
        xa = xa_ref[b]
        ya = ya_ref[b]
        d = jax.lax.dot_general(
            xa, ya, (((1,), (1,)), ((), ())), preferred_element_type=jnp.float32
        )
        rowmin = jnp.min(d, axis=1)
        colmin = jnp.min(d, axis=0)
        acc += jnp.sum(jnp.maximum(rowmin, 0.0)) * inv_bn
        acc += jnp.sum(jnp.maximum(colmin, 0.0)) * inv_bm
    out_ref[0, 0] = acc


@jax.jit
def kernel(x, y):
    B, N, D = x.shape
    _, M, _ = y.shape
    f32 = jnp.float32

    x = x.astype(f32)
    y = y.astype(f32)
    x2 = jnp.sum(x * x, axis=-1, keepdims=True)
    y2 = jnp.sum(y * y, axis=-1, keepdims=True)
    ones_x = jnp.ones_like(x2)
    ones_y = jnp.ones_like(y2)
    zpad_x = jnp.zeros((B, N, 3), f32)
    zpad_y = jnp.zeros((B, M, 3), f32)
    xa = jnp.concatenate([-2.0 * x, x2, ones_x, zpad_x], axis=-1)
    ya = jnp.concatenate([y, ones_y, y2, zpad_y], axis=-1)

    out = pl.pallas_call(
        functools.partial(
            _chamfer_kernel, nbatch=B, inv_bn=1.0 / (B * N), inv_bm=1.0 / (B * M)
        ),
        in_specs=[
            pl.BlockSpec((B, N, 8), lambda: (0, 0, 0)),
            pl.BlockSpec((B, M, 8), lambda: (0, 0, 0)),
        ],
        out_specs=pl.BlockSpec((1, 1), lambda: (0, 0), memory_space=pltpu.SMEM),
        out_shape=jax.ShapeDtypeStruct((1, 1), f32),
    )(xa, ya)
    return out[0, 0]


# speedup vs baseline: 1.7360x; 1.7360x over previous
import functools

import jax
import jax.numpy as jnp
from jax.experimental import pallas as pl
from jax.experimental.pallas import tpu as pltpu


def _chamfer_kernel(xa_ref, ya_ref, out_ref, *, nbatch, inv_bn, inv_bm):
    acc = jnp.float32(0.0)
    for b in range(nbatch):
        xa = xa_ref[b]
        ya = ya_ref[b]
        d = jax.lax.dot_general(
            xa, ya, (((1,), (1,)), ((), ())), preferred_element_type=jnp.float32
        )
        rowmin = jnp.min(d, axis=1)
        colmin = jnp.min(d, axis=0)
        acc += jnp.sum(jnp.maximum(rowmin, 0.0)) * inv_bn
        acc += jnp.sum(jnp.maximum(colmin, 0.0)) * inv_bm
    out_ref[0, 0] = acc


@jax.jit
def kernel(x, y):
    B, N, D = x.shape
    _, M, _ = y.shape
    f32 = jnp.float32

    x = x.astype(f32)
    y = y.astype(f32)
    x2 = jnp.sum(x * x, axis=-1, keepdims=True)
    y2 = jnp.sum(y * y, axis=-1, keepdims=True)
    ones_x = jnp.ones_like(x2)
    ones_y = jnp.ones_like(y2)
    zpad_x = jnp.zeros((B, N, 3), f32)
    zpad_y = jnp.zeros((B, M, 3), f32)
    xa = jnp.concatenate([-2.0 * x, x2, ones_x, zpad_x], axis=-1)
    ya = jnp.concatenate([y, ones_y, y2, zpad_y], axis=-1)

    bf16 = jnp.bfloat16

    def _split3(a):
        hi = a.astype(bf16)
        r1 = a - hi.astype(f32)
        mid = r1.astype(bf16)
        lo = (r1 - mid.astype(f32)).astype(bf16)
        return hi, mid, lo

    xhi, xmid, xlo = _split3(xa)
    yhi, ymid, ylo = _split3(ya)
    xs = jnp.concatenate([xhi, xhi, xhi, xmid, xmid, xlo], axis=-1)
    ys = jnp.concatenate([yhi, ymid, ylo, yhi, ymid, yhi], axis=-1)

    out = pl.pallas_call(
        functools.partial(
            _chamfer_kernel, nbatch=B, inv_bn=1.0 / (B * N), inv_bm=1.0 / (B * M)
        ),
        in_specs=[
            pl.BlockSpec((B, N, 48), lambda: (0, 0, 0)),
            pl.BlockSpec((B, M, 48), lambda: (0, 0, 0)),
        ],
        out_specs=pl.BlockSpec((1, 1), lambda: (0, 0), memory_space=pltpu.SMEM),
        out_shape=jax.ShapeDtypeStruct((1, 1), f32),
    )(xs, ys)
    return out[0, 0]
